# baseline (speedup 1.0000x reference)
import functools

import jax
import jax.numpy as jnp
from jax import lax
from jax.experimental import pallas as pl
from jax.experimental.pallas import tpu as pltpu
from jax.experimental.pallas import tpu_sc as plsc

N_NODES = 10000
IN_CH = 256
HID = 512
N_EDGES = 160000
NPAD = 10240

NC = 2
NS = 16
N_CHUNK = 4
CHUNK = HID // N_CHUNK
EPAD = 163840
EB = 80
NB = EPAD // (NS * EB)
NBH = NB // 2
ROWS_PT = NPAD // NS
ZROWS = 128

DEG_EPT = EPAD // (NC * NS)
DEG_EB = 40
DEG_NB = DEG_EPT // DEG_EB

_sc_mesh = plsc.VectorSubcoreMesh(core_axis_name="c", subcore_axis_name="s")


@functools.partial(
    pl.kernel,
    out_type=jax.ShapeDtypeStruct((NC * NPAD, 16), jnp.float32),
    mesh=_sc_mesh,
    scratch_types=[
        pltpu.VMEM_SHARED((NPAD, 16), jnp.float32),
        pltpu.VMEM((DEG_NB, DEG_EB), jnp.int32),
        pltpu.VMEM((DEG_EB, 16), jnp.float32),
        pltpu.VMEM((ZROWS, 16), jnp.float32),
    ],
)
def _deg_kernel(dst_hbm, deg_hbm, acc, idxv, ones, zbuf):
    c = lax.axis_index("c")
    s = lax.axis_index("s")
    wid = c * NS + s

    def fill_ones(i, _):
        ones[i, :] = jnp.full((16,), 1.0, jnp.float32)
        return 0

    def fill_z(i, _):
        zbuf[i, :] = jnp.zeros((16,), jnp.float32)
        return 0

    lax.fori_loop(0, DEG_EB, fill_ones, 0)
    lax.fori_loop(0, ZROWS, fill_z, 0)
    for k in range(ROWS_PT // ZROWS):
        pltpu.sync_copy(zbuf, acc.at[pl.ds(s * ROWS_PT + k * ZROWS, ZROWS)])
    plsc.subcore_barrier()
    pltpu.sync_copy(dst_hbm.at[wid], idxv)

    def body(j, _):
        pltpu.sync_copy(ones, acc.at[idxv.at[j]], add=True)
        return 0

    lax.fori_loop(0, DEG_NB, body, 0)
    plsc.subcore_barrier()
    pltpu.sync_copy(
        acc.at[pl.ds(s * ROWS_PT, ROWS_PT)],
        deg_hbm.at[pl.ds(c * NPAD + s * ROWS_PT, ROWS_PT)],
    )


@functools.partial(
    pl.kernel,
    out_type=jax.ShapeDtypeStruct((N_CHUNK * NPAD, CHUNK), jnp.float32),
    mesh=_sc_mesh,
    scratch_types=[
        pltpu.VMEM_SHARED((NPAD, CHUNK), jnp.float32),
        pltpu.VMEM((NB, EB), jnp.int32),
        pltpu.VMEM((NB, EB), jnp.int32),
        pltpu.VMEM((EB, CHUNK), jnp.float32),
    ],
)
def _scatter_kernel(p_hbm, src2_hbm, dst2_hbm, s_hbm, acc, srcv, dstv, bufa):
    c = lax.axis_index("c")
    s = lax.axis_index("s")

    def fill_z(i, _):
        for k in range(CHUNK // 16):
            bufa[i, pl.ds(k * 16, 16)] = jnp.zeros((16,), jnp.float32)
        return 0

    pltpu.sync_copy(dst2_hbm.at[s], dstv)

    for cc in range(N_CHUNK // NC):
        chunk = cc * NC + c
        lax.fori_loop(0, EB, fill_z, 0)
        for k in range(ROWS_PT // EB):
            pltpu.sync_copy(bufa, acc.at[pl.ds(s * ROWS_PT + k * EB, EB)])
        pltpu.sync_copy(src2_hbm.at[chunk * NS + s], srcv)
        plsc.subcore_barrier()

        def body(j, _):
            pltpu.sync_copy(p_hbm.at[srcv.at[j]], bufa)
            pltpu.sync_copy(bufa, acc.at[dstv.at[j]], add=True)
            return 0

        lax.fori_loop(0, NB, body, 0)
        plsc.subcore_barrier()
        pltpu.sync_copy(
            acc.at[pl.ds(s * ROWS_PT, ROWS_PT)],
            s_hbm.at[pl.ds(chunk * NPAD + s * ROWS_PT, ROWS_PT)],
        )
        plsc.subcore_barrier()


def _mm_body(h_ref, w_ref, deg_ref, out_ref):
    acc = jnp.dot(h_ref[...], w_ref[...], preferred_element_type=jnp.float32)
    deg = deg_ref[0, :, 0:1] + deg_ref[1, :, 0:1] + 1.0
    p = acc * lax.rsqrt(deg)
    for k in range(N_CHUNK):
        out_ref[k] = p[:, k * CHUNK:(k + 1) * CHUNK]


def _matmul(h, w, deg2):
    m, kdim = h.shape
    bm = 1000
    return pl.pallas_call(
        _mm_body,
        grid=(m // bm,),
        in_specs=[
            pl.BlockSpec((bm, kdim), lambda i: (i, 0)),
            pl.BlockSpec((kdim, HID), lambda i: (0, 0)),
            pl.BlockSpec((NC, bm, 16), lambda i: (0, i, 0)),
        ],
        out_specs=pl.BlockSpec((N_CHUNK, bm, CHUNK), lambda i: (0, i, 0)),
        out_shape=jax.ShapeDtypeStruct((N_CHUNK, NPAD, CHUNK), jnp.float32),
    )(h, w, deg2)


def _epi_body(with_skip, s_ref, p_ref, deg_ref, b_ref, *rest):
    if with_skip:
        y_ref, out_ref = rest
    else:
        (out_ref,) = rest
    deg = deg_ref[0, :, 0:1] + deg_ref[1, :, 0:1] + 1.0
    dinv = lax.rsqrt(deg)
    parts = []
    for k in range(N_CHUNK):
        t = dinv * (s_ref[k] + p_ref[k]) + b_ref[0:1, k * CHUNK:(k + 1) * CHUNK]
        parts.append(t)
    t = jnp.concatenate(parts, axis=1)
    if with_skip:
        t = t + y_ref[...]
    out_ref[...] = jnp.maximum(t, 0.0)


def _epilogue(s4, p4, deg2, b, y_prev=None):
    bm = 1000
    m = N_NODES
    with_skip = y_prev is not None
    in_specs = [
        pl.BlockSpec((N_CHUNK, bm, CHUNK), lambda i: (0, i, 0)),
        pl.BlockSpec((N_CHUNK, bm, CHUNK), lambda i: (0, i, 0)),
        pl.BlockSpec((NC, bm, 16), lambda i: (0, i, 0)),
        pl.BlockSpec((1, HID), lambda i: (0, 0)),
    ]
    args = [s4, p4, deg2, b]
    if with_skip:
        in_specs.append(pl.BlockSpec((bm, HID), lambda i: (i, 0)))
        args.append(y_prev)
    return pl.pallas_call(
        functools.partial(_epi_body, with_skip),
        grid=(m // bm,),
        in_specs=in_specs,
        out_specs=pl.BlockSpec((bm, HID), lambda i: (i, 0)),
        out_shape=jax.ShapeDtypeStruct((m, HID), jnp.float32),
    )(*args)


def kernel(x, edge_index, W0, b0, W1, b1, W2, b2):
    src = edge_index[0].astype(jnp.int32)
    dst = edge_index[1].astype(jnp.int32)

    npad_e = EPAD - N_EDGES
    srcp = jnp.concatenate([src, jnp.zeros((npad_e,), jnp.int32)])
    pad_dst = N_NODES + (jnp.arange(npad_e, dtype=jnp.int32) % (NPAD - N_NODES))
    dstp = jnp.concatenate([dst, pad_dst])
    src2 = (srcp[None, :] + (jnp.arange(N_CHUNK, dtype=jnp.int32) * NPAD)[:, None])
    src2 = src2.reshape(N_CHUNK * NS, NB, EB)
    dst2 = dstp.reshape(NS, NB, EB)
    dst_deg = dstp.reshape(NC * NS, DEG_NB, DEG_EB)

    deg = _deg_kernel(dst_deg)
    deg2 = deg.reshape(NC, NPAD, 16)

    def conv(h_in, w, b, y_prev=None):
        p4 = _matmul(h_in, w, deg2)
        s4 = _scatter_kernel(p4.reshape(N_CHUNK * NPAD, CHUNK), src2, dst2)
        return _epilogue(s4.reshape(N_CHUNK, NPAD, CHUNK), p4, deg2,
                         b.reshape(1, HID), y_prev)

    y0 = conv(x, W0, b0)
    skips = []
    for i in range(3):
        t = conv(y0, W1[i], b1[i])
        y0 = conv(t, W2[i], b2[i], y_prev=y0)
        skips.append(y0)
    return tuple(skips)


# speedup vs baseline: 1.6416x; 1.6416x over previous
import functools

import jax
import jax.numpy as jnp
from jax import lax
from jax.experimental import pallas as pl
from jax.experimental.pallas import tpu as pltpu
from jax.experimental.pallas import tpu_sc as plsc

N_NODES = 10000
IN_CH = 256
HID = 512
N_EDGES = 160000
NPAD = 10240

NC = 2
NS = 16
N_CHUNK = 4
CHUNK = HID // N_CHUNK
EPAD = 160000
EB = 80
NB = EPAD // (NS * EB)
NBH = NB // 2
ROWS_PT = NPAD // NS
ZROWS = 128

DEG_EPT = EPAD // (NC * NS)
DEG_EB = 40
DEG_NB = DEG_EPT // DEG_EB

_sc_mesh = plsc.VectorSubcoreMesh(core_axis_name="c", subcore_axis_name="s")


@functools.partial(
    pl.kernel,
    out_type=jax.ShapeDtypeStruct((NC * NPAD, 16), jnp.float32),
    mesh=_sc_mesh,
    scratch_types=[
        pltpu.VMEM_SHARED((NPAD, 16), jnp.float32),
        pltpu.VMEM((DEG_NB, DEG_EB), jnp.int32),
        pltpu.VMEM((DEG_EB, 16), jnp.float32),
        pltpu.VMEM((ZROWS, 16), jnp.float32),
    ],
)
def _deg_kernel(dst_hbm, deg_hbm, acc, idxv, ones, zbuf):
    c = lax.axis_index("c")
    s = lax.axis_index("s")
    wid = c * NS + s

    def fill_ones(i, _):
        ones[i, :] = jnp.full((16,), 1.0, jnp.float32)
        return 0

    def fill_z(i, _):
        zbuf[i, :] = jnp.zeros((16,), jnp.float32)
        return 0

    lax.fori_loop(0, DEG_EB, fill_ones, 0)
    lax.fori_loop(0, ZROWS, fill_z, 0)
    for k in range(ROWS_PT // ZROWS):
        pltpu.sync_copy(zbuf, acc.at[pl.ds(s * ROWS_PT + k * ZROWS, ZROWS)])
    plsc.subcore_barrier()
    pltpu.sync_copy(dst_hbm.at[wid], idxv)

    def body(j, _):
        pltpu.sync_copy(ones, acc.at[idxv.at[j]], add=True)
        return 0

    lax.fori_loop(0, DEG_NB, body, 0)
    plsc.subcore_barrier()
    pltpu.sync_copy(
        acc.at[pl.ds(s * ROWS_PT, ROWS_PT)],
        deg_hbm.at[pl.ds(c * NPAD + s * ROWS_PT, ROWS_PT)],
    )


@functools.partial(
    pl.kernel,
    out_type=jax.ShapeDtypeStruct((N_CHUNK * NPAD, CHUNK), jnp.float32),
    mesh=_sc_mesh,
    scratch_types=[
        pltpu.VMEM_SHARED((NPAD, CHUNK), jnp.float32),
        pltpu.VMEM((NB, EB), jnp.int32),
        pltpu.VMEM((NB, EB), jnp.int32),
        pltpu.VMEM((EB, CHUNK), jnp.float32),
    ],
)
def _scatter_kernel(p_hbm, src2_hbm, dst2_hbm, s_hbm, acc, srcv, dstv, bufa):
    c = lax.axis_index("c")
    s = lax.axis_index("s")

    def fill_z(i, _):
        for k in range(CHUNK // 16):
            bufa[i, pl.ds(k * 16, 16)] = jnp.zeros((16,), jnp.float32)
        return 0

    pltpu.sync_copy(dst2_hbm.at[s], dstv)

    for cc in range(N_CHUNK // NC):
        chunk = cc * NC + c
        lax.fori_loop(0, EB, fill_z, 0)
        for k in range(ROWS_PT // EB):
            pltpu.sync_copy(bufa, acc.at[pl.ds(s * ROWS_PT + k * EB, EB)])
        pltpu.sync_copy(src2_hbm.at[chunk * NS + s], srcv)
        plsc.subcore_barrier()

        def body(j, _):
            pltpu.sync_copy(p_hbm.at[srcv.at[j]], bufa)
            pltpu.sync_copy(bufa, acc.at[dstv.at[j]], add=True)
            return 0

        lax.fori_loop(0, NB, body, 0)
        plsc.subcore_barrier()
        pltpu.sync_copy(
            acc.at[pl.ds(s * ROWS_PT, ROWS_PT)],
            s_hbm.at[pl.ds(chunk * NPAD + s * ROWS_PT, ROWS_PT)],
        )
        plsc.subcore_barrier()


def _mm_body(h_ref, w_ref, deg_ref, out_ref):
    acc = jnp.dot(h_ref[...], w_ref[...], preferred_element_type=jnp.float32)
    deg = deg_ref[0, :, 0:1] + deg_ref[1, :, 0:1] + 1.0
    p = acc * lax.rsqrt(deg)
    for k in range(N_CHUNK):
        out_ref[k] = p[:, k * CHUNK:(k + 1) * CHUNK]


def _matmul(h, w, deg2):
    m, kdim = h.shape
    bm = 1000
    return pl.pallas_call(
        _mm_body,
        grid=(m // bm,),
        in_specs=[
            pl.BlockSpec((bm, kdim), lambda i: (i, 0)),
            pl.BlockSpec((kdim, HID), lambda i: (0, 0)),
            pl.BlockSpec((NC, bm, 16), lambda i: (0, i, 0)),
        ],
        out_specs=pl.BlockSpec((N_CHUNK, bm, CHUNK), lambda i: (0, i, 0)),
        out_shape=jax.ShapeDtypeStruct((N_CHUNK, NPAD, CHUNK), jnp.float32),
    )(h, w, deg2)


def _epi_body(with_skip, s_ref, p_ref, deg_ref, b_ref, *rest):
    if with_skip:
        y_ref, out_ref = rest
    else:
        (out_ref,) = rest
    deg = deg_ref[0, :, 0:1] + deg_ref[1, :, 0:1] + 1.0
    dinv = lax.rsqrt(deg)
    parts = []
    for k in range(N_CHUNK):
        t = dinv * (s_ref[k] + p_ref[k]) + b_ref[0:1, k * CHUNK:(k + 1) * CHUNK]
        parts.append(t)
    t = jnp.concatenate(parts, axis=1)
    if with_skip:
        t = t + y_ref[...]
    out_ref[...] = jnp.maximum(t, 0.0)


def _epilogue(s4, p4, deg2, b, y_prev=None):
    bm = 1000
    m = N_NODES
    with_skip = y_prev is not None
    in_specs = [
        pl.BlockSpec((N_CHUNK, bm, CHUNK), lambda i: (0, i, 0)),
        pl.BlockSpec((N_CHUNK, bm, CHUNK), lambda i: (0, i, 0)),
        pl.BlockSpec((NC, bm, 16), lambda i: (0, i, 0)),
        pl.BlockSpec((1, HID), lambda i: (0, 0)),
    ]
    args = [s4, p4, deg2, b]
    if with_skip:
        in_specs.append(pl.BlockSpec((bm, HID), lambda i: (i, 0)))
        args.append(y_prev)
    return pl.pallas_call(
        functools.partial(_epi_body, with_skip),
        grid=(m // bm,),
        in_specs=in_specs,
        out_specs=pl.BlockSpec((bm, HID), lambda i: (i, 0)),
        out_shape=jax.ShapeDtypeStruct((m, HID), jnp.float32),
    )(*args)


def kernel(x, edge_index, W0, b0, W1, b1, W2, b2):
    src = edge_index[0].astype(jnp.int32)
    dst = edge_index[1].astype(jnp.int32)

    npad_e = EPAD - N_EDGES
    srcp = jnp.concatenate([src, jnp.zeros((npad_e,), jnp.int32)])
    pad_dst = N_NODES + (jnp.arange(npad_e, dtype=jnp.int32) % (NPAD - N_NODES))
    dstp = jnp.concatenate([dst, pad_dst])
    src2 = (srcp[None, :] + (jnp.arange(N_CHUNK, dtype=jnp.int32) * NPAD)[:, None])
    src2 = src2.reshape(N_CHUNK * NS, NB, EB)
    dst2 = dstp.reshape(NS, NB, EB)
    dst_deg = dstp.reshape(NC * NS, DEG_NB, DEG_EB)

    deg = _deg_kernel(dst_deg)
    deg2 = deg.reshape(NC, NPAD, 16)

    def conv(h_in, w, b, y_prev=None):
        p4 = _matmul(h_in, w, deg2)
        s4 = _scatter_kernel(p4.reshape(N_CHUNK * NPAD, CHUNK), src2, dst2)
        return _epilogue(s4.reshape(N_CHUNK, NPAD, CHUNK), p4, deg2,
                         b.reshape(1, HID), y_prev)

    y0 = conv(x, W0, b0)
    skips = []
    for i in range(3):
        t = conv(y0, W1[i], b1[i])
        y0 = conv(t, W2[i], b2[i], y_prev=y0)
        skips.append(y0)
    return tuple(skips)


# speedup vs baseline: 1.7807x; 1.0847x over previous
import functools

import jax
import jax.numpy as jnp
from jax import lax
from jax.experimental import pallas as pl
from jax.experimental.pallas import tpu as pltpu
from jax.experimental.pallas import tpu_sc as plsc

N_NODES = 10000
IN_CH = 256
HID = 512
N_EDGES = 160000
NPAD = 10240

NC = 2
NS = 16
N_CHUNK = 4
CHUNK = HID // N_CHUNK
EPAD = 160000
EB = 100
NB = EPAD // (NS * EB)
NBH = NB // 2
ROWS_PT = NPAD // NS
ZROWS = 128

DEG_EPT = EPAD // (NC * NS)
DEG_EB = 40
DEG_NB = DEG_EPT // DEG_EB

_sc_mesh = plsc.VectorSubcoreMesh(core_axis_name="c", subcore_axis_name="s")


@functools.partial(
    pl.kernel,
    out_type=jax.ShapeDtypeStruct((NC * NPAD, 16), jnp.float32),
    mesh=_sc_mesh,
    scratch_types=[
        pltpu.VMEM_SHARED((NPAD, 16), jnp.float32),
        pltpu.VMEM((DEG_NB, DEG_EB), jnp.int32),
        pltpu.VMEM((DEG_EB, 16), jnp.float32),
        pltpu.VMEM((ZROWS, 16), jnp.float32),
    ],
)
def _deg_kernel(dst_hbm, deg_hbm, acc, idxv, ones, zbuf):
    c = lax.axis_index("c")
    s = lax.axis_index("s")
    wid = c * NS + s

    def fill_ones(i, _):
        ones[i, :] = jnp.full((16,), 1.0, jnp.float32)
        return 0

    def fill_z(i, _):
        zbuf[i, :] = jnp.zeros((16,), jnp.float32)
        return 0

    lax.fori_loop(0, DEG_EB, fill_ones, 0)
    lax.fori_loop(0, ZROWS, fill_z, 0)
    for k in range(ROWS_PT // ZROWS):
        pltpu.sync_copy(zbuf, acc.at[pl.ds(s * ROWS_PT + k * ZROWS, ZROWS)])
    plsc.subcore_barrier()
    pltpu.sync_copy(dst_hbm.at[wid], idxv)

    def body(j, _):
        pltpu.sync_copy(ones, acc.at[idxv.at[j]], add=True)
        return 0

    lax.fori_loop(0, DEG_NB, body, 0)
    plsc.subcore_barrier()
    pltpu.sync_copy(
        acc.at[pl.ds(s * ROWS_PT, ROWS_PT)],
        deg_hbm.at[pl.ds(c * NPAD + s * ROWS_PT, ROWS_PT)],
    )


@functools.partial(
    pl.kernel,
    out_type=jax.ShapeDtypeStruct((N_CHUNK * NPAD, CHUNK), jnp.float32),
    mesh=_sc_mesh,
    scratch_types=[
        pltpu.VMEM_SHARED((NPAD, CHUNK), jnp.float32),
        pltpu.VMEM((NB, EB), jnp.int32),
        pltpu.VMEM((NB, EB), jnp.int32),
        pltpu.VMEM((EB, CHUNK), jnp.float32),
    ],
)
def _scatter_kernel(p_hbm, src2_hbm, dst2_hbm, s_hbm, acc, srcv, dstv, bufa):
    c = lax.axis_index("c")
    s = lax.axis_index("s")

    def fill_z(i, _):
        for k in range(CHUNK // 16):
            bufa[i, pl.ds(k * 16, 16)] = jnp.zeros((16,), jnp.float32)
        return 0

    pltpu.sync_copy(dst2_hbm.at[s], dstv)

    for cc in range(N_CHUNK // NC):
        chunk = cc * NC + c
        lax.fori_loop(0, EB, fill_z, 0)
        for k in range(ROWS_PT // EB):
            pltpu.sync_copy(bufa, acc.at[pl.ds(s * ROWS_PT + k * EB, EB)])
        pltpu.sync_copy(src2_hbm.at[chunk * NS + s], srcv)
        plsc.subcore_barrier()

        def body(j, _):
            pltpu.sync_copy(p_hbm.at[srcv.at[j]], bufa)
            pltpu.sync_copy(bufa, acc.at[dstv.at[j]], add=True)
            return 0

        lax.fori_loop(0, NB, body, 0)
        plsc.subcore_barrier()
        pltpu.sync_copy(
            acc.at[pl.ds(s * ROWS_PT, ROWS_PT)],
            s_hbm.at[pl.ds(chunk * NPAD + s * ROWS_PT, ROWS_PT)],
        )
        plsc.subcore_barrier()


def _mm_body(h_ref, w_ref, deg_ref, out_ref):
    acc = jnp.dot(h_ref[...], w_ref[...], preferred_element_type=jnp.float32)
    deg = deg_ref[0, :, 0:1] + deg_ref[1, :, 0:1] + 1.0
    p = acc * lax.rsqrt(deg)
    for k in range(N_CHUNK):
        out_ref[k] = p[:, k * CHUNK:(k + 1) * CHUNK]


def _matmul(h, w, deg2):
    m, kdim = h.shape
    bm = 1000
    return pl.pallas_call(
        _mm_body,
        grid=(m // bm,),
        in_specs=[
            pl.BlockSpec((bm, kdim), lambda i: (i, 0)),
            pl.BlockSpec((kdim, HID), lambda i: (0, 0)),
            pl.BlockSpec((NC, bm, 16), lambda i: (0, i, 0)),
        ],
        out_specs=pl.BlockSpec((N_CHUNK, bm, CHUNK), lambda i: (0, i, 0)),
        out_shape=jax.ShapeDtypeStruct((N_CHUNK, NPAD, CHUNK), jnp.float32),
    )(h, w, deg2)


def _epi_body(with_skip, s_ref, p_ref, deg_ref, b_ref, *rest):
    if with_skip:
        y_ref, out_ref = rest
    else:
        (out_ref,) = rest
    deg = deg_ref[0, :, 0:1] + deg_ref[1, :, 0:1] + 1.0
    dinv = lax.rsqrt(deg)
    parts = []
    for k in range(N_CHUNK):
        t = dinv * (s_ref[k] + p_ref[k]) + b_ref[0:1, k * CHUNK:(k + 1) * CHUNK]
        parts.append(t)
    t = jnp.concatenate(parts, axis=1)
    if with_skip:
        t = t + y_ref[...]
    out_ref[...] = jnp.maximum(t, 0.0)


def _epilogue(s4, p4, deg2, b, y_prev=None):
    bm = 1000
    m = N_NODES
    with_skip = y_prev is not None
    in_specs = [
        pl.BlockSpec((N_CHUNK, bm, CHUNK), lambda i: (0, i, 0)),
        pl.BlockSpec((N_CHUNK, bm, CHUNK), lambda i: (0, i, 0)),
        pl.BlockSpec((NC, bm, 16), lambda i: (0, i, 0)),
        pl.BlockSpec((1, HID), lambda i: (0, 0)),
    ]
    args = [s4, p4, deg2, b]
    if with_skip:
        in_specs.append(pl.BlockSpec((bm, HID), lambda i: (i, 0)))
        args.append(y_prev)
    return pl.pallas_call(
        functools.partial(_epi_body, with_skip),
        grid=(m // bm,),
        in_specs=in_specs,
        out_specs=pl.BlockSpec((bm, HID), lambda i: (i, 0)),
        out_shape=jax.ShapeDtypeStruct((m, HID), jnp.float32),
    )(*args)


def kernel(x, edge_index, W0, b0, W1, b1, W2, b2):
    src = edge_index[0].astype(jnp.int32)
    dst = edge_index[1].astype(jnp.int32)

    npad_e = EPAD - N_EDGES
    srcp = jnp.concatenate([src, jnp.zeros((npad_e,), jnp.int32)])
    pad_dst = N_NODES + (jnp.arange(npad_e, dtype=jnp.int32) % (NPAD - N_NODES))
    dstp = jnp.concatenate([dst, pad_dst])
    src2 = (srcp[None, :] + (jnp.arange(N_CHUNK, dtype=jnp.int32) * NPAD)[:, None])
    src2 = src2.reshape(N_CHUNK * NS, NB, EB)
    dst2 = dstp.reshape(NS, NB, EB)
    dst_deg = dstp.reshape(NC * NS, DEG_NB, DEG_EB)

    deg = _deg_kernel(dst_deg)
    deg2 = deg.reshape(NC, NPAD, 16)

    def conv(h_in, w, b, y_prev=None):
        p4 = _matmul(h_in, w, deg2)
        s4 = _scatter_kernel(p4.reshape(N_CHUNK * NPAD, CHUNK), src2, dst2)
        return _epilogue(s4.reshape(N_CHUNK, NPAD, CHUNK), p4, deg2,
                         b.reshape(1, HID), y_prev)

    y0 = conv(x, W0, b0)
    skips = []
    for i in range(3):
        t = conv(y0, W1[i], b1[i])
        y0 = conv(t, W2[i], b2[i], y_prev=y0)
        skips.append(y0)
    return tuple(skips)


# speedup vs baseline: 1.9037x; 1.0691x over previous
import functools

import jax
import jax.numpy as jnp
from jax import lax
from jax.experimental import pallas as pl
from jax.experimental.pallas import tpu as pltpu
from jax.experimental.pallas import tpu_sc as plsc

N_NODES = 10000
IN_CH = 256
HID = 512
N_EDGES = 160000
NPAD = 10240

NC = 2
NS = 16
N_CHUNK = 4
CHUNK = HID // N_CHUNK
EPAD = 160000
EB = 125
NB = EPAD // (NS * EB)
NBH = NB // 2
ROWS_PT = NPAD // NS
ZROWS = 128

DEG_EPT = EPAD // (NC * NS)
DEG_EB = 40
DEG_NB = DEG_EPT // DEG_EB

_sc_mesh = plsc.VectorSubcoreMesh(core_axis_name="c", subcore_axis_name="s")


@functools.partial(
    pl.kernel,
    out_type=jax.ShapeDtypeStruct((NC * NPAD, 16), jnp.float32),
    mesh=_sc_mesh,
    scratch_types=[
        pltpu.VMEM_SHARED((NPAD, 16), jnp.float32),
        pltpu.VMEM((DEG_NB, DEG_EB), jnp.int32),
        pltpu.VMEM((DEG_EB, 16), jnp.float32),
        pltpu.VMEM((ZROWS, 16), jnp.float32),
    ],
)
def _deg_kernel(dst_hbm, deg_hbm, acc, idxv, ones, zbuf):
    c = lax.axis_index("c")
    s = lax.axis_index("s")
    wid = c * NS + s

    def fill_ones(i, _):
        ones[i, :] = jnp.full((16,), 1.0, jnp.float32)
        return 0

    def fill_z(i, _):
        zbuf[i, :] = jnp.zeros((16,), jnp.float32)
        return 0

    lax.fori_loop(0, DEG_EB, fill_ones, 0)
    lax.fori_loop(0, ZROWS, fill_z, 0)
    for k in range(ROWS_PT // ZROWS):
        pltpu.sync_copy(zbuf, acc.at[pl.ds(s * ROWS_PT + k * ZROWS, ZROWS)])
    plsc.subcore_barrier()
    pltpu.sync_copy(dst_hbm.at[wid], idxv)

    def body(j, _):
        pltpu.sync_copy(ones, acc.at[idxv.at[j]], add=True)
        return 0

    lax.fori_loop(0, DEG_NB, body, 0)
    plsc.subcore_barrier()
    pltpu.sync_copy(
        acc.at[pl.ds(s * ROWS_PT, ROWS_PT)],
        deg_hbm.at[pl.ds(c * NPAD + s * ROWS_PT, ROWS_PT)],
    )


@functools.partial(
    pl.kernel,
    out_type=jax.ShapeDtypeStruct((N_CHUNK * NPAD, CHUNK), jnp.float32),
    mesh=_sc_mesh,
    scratch_types=[
        pltpu.VMEM_SHARED((NPAD, CHUNK), jnp.float32),
        pltpu.VMEM((NB, EB), jnp.int32),
        pltpu.VMEM((NB, EB), jnp.int32),
        pltpu.VMEM((EB, CHUNK), jnp.float32),
    ],
)
def _scatter_kernel(p_hbm, src2_hbm, dst2_hbm, s_hbm, acc, srcv, dstv, bufa):
    c = lax.axis_index("c")
    s = lax.axis_index("s")

    def fill_z(i, _):
        for k in range(CHUNK // 16):
            bufa[i, pl.ds(k * 16, 16)] = jnp.zeros((16,), jnp.float32)
        return 0

    pltpu.sync_copy(dst2_hbm.at[s], dstv)

    for cc in range(N_CHUNK // NC):
        chunk = cc * NC + c
        lax.fori_loop(0, EB, fill_z, 0)
        for k in range(ROWS_PT // EB):
            pltpu.sync_copy(bufa, acc.at[pl.ds(s * ROWS_PT + k * EB, EB)])
        pltpu.sync_copy(src2_hbm.at[chunk * NS + s], srcv)
        plsc.subcore_barrier()

        def body(j, _):
            pltpu.sync_copy(p_hbm.at[srcv.at[j]], bufa)
            pltpu.sync_copy(bufa, acc.at[dstv.at[j]], add=True)
            return 0

        lax.fori_loop(0, NB, body, 0)
        plsc.subcore_barrier()
        pltpu.sync_copy(
            acc.at[pl.ds(s * ROWS_PT, ROWS_PT)],
            s_hbm.at[pl.ds(chunk * NPAD + s * ROWS_PT, ROWS_PT)],
        )
        plsc.subcore_barrier()


def _mm_body(h_ref, w_ref, deg_ref, out_ref):
    acc = jnp.dot(h_ref[...], w_ref[...], preferred_element_type=jnp.float32)
    deg = deg_ref[0, :, 0:1] + deg_ref[1, :, 0:1] + 1.0
    p = acc * lax.rsqrt(deg)
    for k in range(N_CHUNK):
        out_ref[k] = p[:, k * CHUNK:(k + 1) * CHUNK]


def _matmul(h, w, deg2):
    m, kdim = h.shape
    bm = 1000
    return pl.pallas_call(
        _mm_body,
        grid=(m // bm,),
        in_specs=[
            pl.BlockSpec((bm, kdim), lambda i: (i, 0)),
            pl.BlockSpec((kdim, HID), lambda i: (0, 0)),
            pl.BlockSpec((NC, bm, 16), lambda i: (0, i, 0)),
        ],
        out_specs=pl.BlockSpec((N_CHUNK, bm, CHUNK), lambda i: (0, i, 0)),
        out_shape=jax.ShapeDtypeStruct((N_CHUNK, NPAD, CHUNK), jnp.float32),
    )(h, w, deg2)


def _epi_body(with_skip, s_ref, p_ref, deg_ref, b_ref, *rest):
    if with_skip:
        y_ref, out_ref = rest
    else:
        (out_ref,) = rest
    deg = deg_ref[0, :, 0:1] + deg_ref[1, :, 0:1] + 1.0
    dinv = lax.rsqrt(deg)
    parts = []
    for k in range(N_CHUNK):
        t = dinv * (s_ref[k] + p_ref[k]) + b_ref[0:1, k * CHUNK:(k + 1) * CHUNK]
        parts.append(t)
    t = jnp.concatenate(parts, axis=1)
    if with_skip:
        t = t + y_ref[...]
    out_ref[...] = jnp.maximum(t, 0.0)


def _epilogue(s4, p4, deg2, b, y_prev=None):
    bm = 1000
    m = N_NODES
    with_skip = y_prev is not None
    in_specs = [
        pl.BlockSpec((N_CHUNK, bm, CHUNK), lambda i: (0, i, 0)),
        pl.BlockSpec((N_CHUNK, bm, CHUNK), lambda i: (0, i, 0)),
        pl.BlockSpec((NC, bm, 16), lambda i: (0, i, 0)),
        pl.BlockSpec((1, HID), lambda i: (0, 0)),
    ]
    args = [s4, p4, deg2, b]
    if with_skip:
        in_specs.append(pl.BlockSpec((bm, HID), lambda i: (i, 0)))
        args.append(y_prev)
    return pl.pallas_call(
        functools.partial(_epi_body, with_skip),
        grid=(m // bm,),
        in_specs=in_specs,
        out_specs=pl.BlockSpec((bm, HID), lambda i: (i, 0)),
        out_shape=jax.ShapeDtypeStruct((m, HID), jnp.float32),
    )(*args)


def kernel(x, edge_index, W0, b0, W1, b1, W2, b2):
    src = edge_index[0].astype(jnp.int32)
    dst = edge_index[1].astype(jnp.int32)

    npad_e = EPAD - N_EDGES
    srcp = jnp.concatenate([src, jnp.zeros((npad_e,), jnp.int32)])
    pad_dst = N_NODES + (jnp.arange(npad_e, dtype=jnp.int32) % (NPAD - N_NODES))
    dstp = jnp.concatenate([dst, pad_dst])
    src2 = (srcp[None, :] + (jnp.arange(N_CHUNK, dtype=jnp.int32) * NPAD)[:, None])
    src2 = src2.reshape(N_CHUNK * NS, NB, EB)
    dst2 = dstp.reshape(NS, NB, EB)
    dst_deg = dstp.reshape(NC * NS, DEG_NB, DEG_EB)

    deg = _deg_kernel(dst_deg)
    deg2 = deg.reshape(NC, NPAD, 16)

    def conv(h_in, w, b, y_prev=None):
        p4 = _matmul(h_in, w, deg2)
        s4 = _scatter_kernel(p4.reshape(N_CHUNK * NPAD, CHUNK), src2, dst2)
        return _epilogue(s4.reshape(N_CHUNK, NPAD, CHUNK), p4, deg2,
                         b.reshape(1, HID), y_prev)

    y0 = conv(x, W0, b0)
    skips = []
    for i in range(3):
        t = conv(y0, W1[i], b1[i])
        y0 = conv(t, W2[i], b2[i], y_prev=y0)
        skips.append(y0)
    return tuple(skips)


# speedup vs baseline: 1.9467x; 1.0226x over previous
import functools

import jax
import jax.numpy as jnp
from jax import lax
from jax.experimental import pallas as pl
from jax.experimental.pallas import tpu as pltpu
from jax.experimental.pallas import tpu_sc as plsc

N_NODES = 10000
IN_CH = 256
HID = 512
N_EDGES = 160000
NPAD = 10240

NC = 2
NS = 16
N_CHUNK = 4
CHUNK = HID // N_CHUNK
EPAD = 160000
EB = 125
NB = EPAD // (NS * EB)
NBH = NB // 2
ROWS_PT = NPAD // NS
ZROWS = 128

DEG_EPT = EPAD // (NC * NS)
DEG_EB = 40
DEG_NB = DEG_EPT // DEG_EB

_sc_mesh = plsc.VectorSubcoreMesh(core_axis_name="c", subcore_axis_name="s")


@functools.partial(
    pl.kernel,
    out_type=jax.ShapeDtypeStruct((NC * NPAD, 16), jnp.float32),
    mesh=_sc_mesh,
    scratch_types=[
        pltpu.VMEM_SHARED((NPAD, 16), jnp.float32),
        pltpu.VMEM((DEG_NB, DEG_EB), jnp.int32),
        pltpu.VMEM((DEG_EB, 16), jnp.float32),
        pltpu.VMEM((ZROWS, 16), jnp.float32),
    ],
)
def _deg_kernel(dst_hbm, deg_hbm, acc, idxv, ones, zbuf):
    c = lax.axis_index("c")
    s = lax.axis_index("s")
    wid = c * NS + s

    def fill_ones(i, _):
        ones[i, :] = jnp.full((16,), 1.0, jnp.float32)
        return 0

    def fill_z(i, _):
        zbuf[i, :] = jnp.zeros((16,), jnp.float32)
        return 0

    lax.fori_loop(0, DEG_EB, fill_ones, 0)
    lax.fori_loop(0, ZROWS, fill_z, 0)
    for k in range(ROWS_PT // ZROWS):
        pltpu.sync_copy(zbuf, acc.at[pl.ds(s * ROWS_PT + k * ZROWS, ZROWS)])
    plsc.subcore_barrier()
    pltpu.sync_copy(dst_hbm.at[wid], idxv)

    def body(j, _):
        pltpu.sync_copy(ones, acc.at[idxv.at[j]], add=True)
        return 0

    lax.fori_loop(0, DEG_NB, body, 0)
    plsc.subcore_barrier()
    pltpu.sync_copy(
        acc.at[pl.ds(s * ROWS_PT, ROWS_PT)],
        deg_hbm.at[pl.ds(c * NPAD + s * ROWS_PT, ROWS_PT)],
    )


@functools.partial(
    pl.kernel,
    out_type=jax.ShapeDtypeStruct((N_CHUNK * NPAD, CHUNK), jnp.float32),
    mesh=_sc_mesh,
    scratch_types=[
        pltpu.VMEM_SHARED((NPAD, CHUNK), jnp.float32),
        pltpu.VMEM((NB, EB), jnp.int32),
        pltpu.VMEM((NB, EB), jnp.int32),
        pltpu.VMEM((EB, CHUNK), jnp.float32),
    ],
)
def _scatter_kernel(p_hbm, src2_hbm, dst2_hbm, s_hbm, acc, srcv, dstv, bufa):
    c = lax.axis_index("c")
    s = lax.axis_index("s")

    def fill_z(i, _):
        for k in range(CHUNK // 16):
            bufa[i, pl.ds(k * 16, 16)] = jnp.zeros((16,), jnp.float32)
        return 0

    pltpu.sync_copy(dst2_hbm.at[s], dstv)

    for cc in range(N_CHUNK // NC):
        chunk = cc * NC + c
        lax.fori_loop(0, EB, fill_z, 0)
        for k in range(ROWS_PT // EB):
            pltpu.sync_copy(bufa, acc.at[pl.ds(s * ROWS_PT + k * EB, EB)])
        pltpu.sync_copy(src2_hbm.at[chunk * NS + s], srcv)
        plsc.subcore_barrier()

        def body(j, _):
            pltpu.sync_copy(p_hbm.at[srcv.at[j]], bufa)
            pltpu.sync_copy(bufa, acc.at[dstv.at[j]], add=True)
            return 0

        lax.fori_loop(0, NB, body, 0)
        plsc.subcore_barrier()
        pltpu.sync_copy(
            acc.at[pl.ds(s * ROWS_PT, ROWS_PT)],
            s_hbm.at[pl.ds(chunk * NPAD + s * ROWS_PT, ROWS_PT)],
        )


def _mm_body(h_ref, w_ref, deg_ref, out_ref):
    acc = jnp.dot(h_ref[...], w_ref[...], preferred_element_type=jnp.float32)
    deg = deg_ref[0, :, 0:1] + deg_ref[1, :, 0:1] + 1.0
    p = acc * lax.rsqrt(deg)
    for k in range(N_CHUNK):
        out_ref[k] = p[:, k * CHUNK:(k + 1) * CHUNK]


def _matmul(h, w, deg2):
    m, kdim = h.shape
    bm = 1000
    return pl.pallas_call(
        _mm_body,
        grid=(m // bm,),
        in_specs=[
            pl.BlockSpec((bm, kdim), lambda i: (i, 0)),
            pl.BlockSpec((kdim, HID), lambda i: (0, 0)),
            pl.BlockSpec((NC, bm, 16), lambda i: (0, i, 0)),
        ],
        out_specs=pl.BlockSpec((N_CHUNK, bm, CHUNK), lambda i: (0, i, 0)),
        out_shape=jax.ShapeDtypeStruct((N_CHUNK, NPAD, CHUNK), jnp.float32),
    )(h, w, deg2)


def _mmf_body(s_ref, p_ref, deg_ref, b_ref, w_ref, out_ref):
    deg = deg_ref[0, :, 0:1] + deg_ref[1, :, 0:1] + 1.0
    dinv = lax.rsqrt(deg)
    parts = []
    for k in range(N_CHUNK):
        parts.append(dinv * (s_ref[k] + p_ref[k])
                     + b_ref[0:1, k * CHUNK:(k + 1) * CHUNK])
    t = jnp.maximum(jnp.concatenate(parts, axis=1), 0.0)
    acc = jnp.dot(t, w_ref[...], preferred_element_type=jnp.float32)
    p = acc * dinv
    for k in range(N_CHUNK):
        out_ref[k] = p[:, k * CHUNK:(k + 1) * CHUNK]


def _matmul_fused(s4, p4, deg2, b, w):
    bm = 1000
    return pl.pallas_call(
        _mmf_body,
        grid=(N_NODES // bm,),
        in_specs=[
            pl.BlockSpec((N_CHUNK, bm, CHUNK), lambda i: (0, i, 0)),
            pl.BlockSpec((N_CHUNK, bm, CHUNK), lambda i: (0, i, 0)),
            pl.BlockSpec((NC, bm, 16), lambda i: (0, i, 0)),
            pl.BlockSpec((1, HID), lambda i: (0, 0)),
            pl.BlockSpec((HID, HID), lambda i: (0, 0)),
        ],
        out_specs=pl.BlockSpec((N_CHUNK, bm, CHUNK), lambda i: (0, i, 0)),
        out_shape=jax.ShapeDtypeStruct((N_CHUNK, NPAD, CHUNK), jnp.float32),
    )(s4, p4, deg2, b, w)


def _epi_body(with_skip, s_ref, p_ref, deg_ref, b_ref, *rest):
    if with_skip:
        y_ref, out_ref = rest
    else:
        (out_ref,) = rest
    deg = deg_ref[0, :, 0:1] + deg_ref[1, :, 0:1] + 1.0
    dinv = lax.rsqrt(deg)
    parts = []
    for k in range(N_CHUNK):
        t = dinv * (s_ref[k] + p_ref[k]) + b_ref[0:1, k * CHUNK:(k + 1) * CHUNK]
        parts.append(t)
    t = jnp.concatenate(parts, axis=1)
    if with_skip:
        t = t + y_ref[...]
    out_ref[...] = jnp.maximum(t, 0.0)


def _epilogue(s4, p4, deg2, b, y_prev=None):
    bm = 1000
    m = N_NODES
    with_skip = y_prev is not None
    in_specs = [
        pl.BlockSpec((N_CHUNK, bm, CHUNK), lambda i: (0, i, 0)),
        pl.BlockSpec((N_CHUNK, bm, CHUNK), lambda i: (0, i, 0)),
        pl.BlockSpec((NC, bm, 16), lambda i: (0, i, 0)),
        pl.BlockSpec((1, HID), lambda i: (0, 0)),
    ]
    args = [s4, p4, deg2, b]
    if with_skip:
        in_specs.append(pl.BlockSpec((bm, HID), lambda i: (i, 0)))
        args.append(y_prev)
    return pl.pallas_call(
        functools.partial(_epi_body, with_skip),
        grid=(m // bm,),
        in_specs=in_specs,
        out_specs=pl.BlockSpec((bm, HID), lambda i: (i, 0)),
        out_shape=jax.ShapeDtypeStruct((m, HID), jnp.float32),
    )(*args)


def kernel(x, edge_index, W0, b0, W1, b1, W2, b2):
    src = edge_index[0].astype(jnp.int32)
    dst = edge_index[1].astype(jnp.int32)

    npad_e = EPAD - N_EDGES
    srcp = jnp.concatenate([src, jnp.zeros((npad_e,), jnp.int32)])
    pad_dst = N_NODES + (jnp.arange(npad_e, dtype=jnp.int32) % (NPAD - N_NODES))
    dstp = jnp.concatenate([dst, pad_dst])
    src2 = (srcp[None, :] + (jnp.arange(N_CHUNK, dtype=jnp.int32) * NPAD)[:, None])
    src2 = src2.reshape(N_CHUNK * NS, NB, EB)
    dst2 = dstp.reshape(NS, NB, EB)
    dst_deg = dstp.reshape(NC * NS, DEG_NB, DEG_EB)

    deg = _deg_kernel(dst_deg)
    deg2 = deg.reshape(NC, NPAD, 16)

    def conv(h_in, w, b, y_prev=None):
        p4 = _matmul(h_in, w, deg2)
        s4 = _scatter_kernel(p4.reshape(N_CHUNK * NPAD, CHUNK), src2, dst2)
        return _epilogue(s4.reshape(N_CHUNK, NPAD, CHUNK), p4, deg2,
                         b.reshape(1, HID), y_prev)

    y0 = conv(x, W0, b0)
    skips = []
    for i in range(3):
        p4t = _matmul(y0, W1[i], deg2)
        s4t = _scatter_kernel(p4t.reshape(N_CHUNK * NPAD, CHUNK), src2, dst2)
        p4y = _matmul_fused(s4t.reshape(N_CHUNK, NPAD, CHUNK), p4t, deg2,
                            b1[i].reshape(1, HID), W2[i])
        s4y = _scatter_kernel(p4y.reshape(N_CHUNK * NPAD, CHUNK), src2, dst2)
        y0 = _epilogue(s4y.reshape(N_CHUNK, NPAD, CHUNK), p4y, deg2,
                       b2[i].reshape(1, HID), y_prev=y0)
        skips.append(y0)
    return tuple(skips)
